# baseline (speedup 1.0000x reference)
import functools

import jax
import jax.numpy as jnp
from jax.experimental import pallas as pl
from jax.experimental.pallas import tpu as pltpu

HEADS = 16
BUCKETS = 64


def _qkv_kern(x_ref, w_ref, o_ref):
    o_ref[0] = jnp.dot(x_ref[...], w_ref[0],
                       preferred_element_type=jnp.float32)


def _sortnet_kern(q_ref, k_ref, ws_ref, idx_ref, w_ref, *, buckets, bsz, d_h):
    q3 = q_ref[0, 0].reshape(buckets, bsz, d_h)
    k3 = k_ref[0, 0].reshape(buckets, bsz, d_h)
    xs = jnp.concatenate((q3.sum(axis=1), k3.sum(axis=1)), axis=-1)
    logits = jnp.dot(xs, ws_ref[0, 0], preferred_element_type=jnp.float32)
    lr = jnp.where(logits >= 0, logits, 0.01 * logits)
    m = jnp.max(lr, axis=-1, keepdims=True)
    sumexp = jnp.sum(jnp.exp(lr - m), axis=-1, keepdims=True)
    col = jax.lax.broadcasted_iota(jnp.int32, (buckets, buckets), 1)
    idx_ref[0] = jnp.min(jnp.where(lr == m, col, buckets),
                         axis=-1, keepdims=True)
    w_ref[0] = 1.0 / sumexp


def _attn_kern(idx_ref, w_ref, q_ref, k_ref, v_ref, o_ref, kg_ref, vg_ref, *,
               heads, buckets, bsz, d_h, scale):
    bh = pl.program_id(0) * heads + pl.program_id(1)
    q3 = q_ref[0, 0].reshape(buckets, bsz, d_h)
    k3 = k_ref[0, 0].reshape(buckets, bsz, d_h)
    v3 = v_ref[0, 0].reshape(buckets, bsz, d_h)

    def gather(u, _):
        g = idx_ref[bh, u]
        w = w_ref[bh, u]
        kg_ref[u] = w * k_ref[0, 0, pl.ds(g * bsz, bsz), :]
        vg_ref[u] = w * v_ref[0, 0, pl.ds(g * bsz, bsz), :]
        return 0

    jax.lax.fori_loop(0, buckets, gather, 0, unroll=True)
    k_r = kg_ref[...]
    v_r = vg_ref[...]
    dn = (((2,), (2,)), ((0,), (0,)))
    dots_g = jax.lax.dot_general(q3, k_r, dn,
                                 preferred_element_type=jnp.float32) * scale
    dots_s = jax.lax.dot_general(q3, k3, dn,
                                 preferred_element_type=jnp.float32) * scale
    dots = jnp.concatenate((dots_g, dots_s), axis=-1)
    m = jnp.max(dots, axis=-1, keepdims=True)
    e = jnp.exp(dots - m)
    p = e / jnp.sum(e, axis=-1, keepdims=True)
    pn = (((2,), (1,)), ((0,), (0,)))
    out = (jax.lax.dot_general(p[:, :, :bsz], v_r, pn,
                               preferred_element_type=jnp.float32)
           + jax.lax.dot_general(p[:, :, bsz:], v3, pn,
                                 preferred_element_type=jnp.float32))
    o_ref[0, 0] = out.reshape(bsz * buckets, d_h)


def _out_kern(x_ref, w_ref, b_ref, o_ref):
    c = pl.program_id(1)

    @pl.when(c == 0)
    def _():
        o_ref[...] = jnp.broadcast_to(b_ref[...], o_ref.shape)

    o_ref[...] += jnp.dot(x_ref[0], w_ref[0],
                          preferred_element_type=jnp.float32)


def kernel(x, W_qkv, W_out, b_out, W_sort):
    b, t, d = x.shape
    h, buckets = HEADS, BUCKETS
    d_h = d // h
    bsz = t // buckets
    f32 = jnp.float32

    W_qkv_r = W_qkv.reshape(d, 3 * h, d_h).transpose(1, 0, 2)
    W_out_r = W_out.reshape(h, d_h, d)
    b_out_r = b_out.reshape(1, d)

    bm = 2048
    qkv = pl.pallas_call(
        _qkv_kern,
        grid=(b * t // bm, 3 * h),
        in_specs=[
            pl.BlockSpec((bm, d), lambda i, j: (i, 0)),
            pl.BlockSpec((1, d, d_h), lambda i, j: (j, 0, 0)),
        ],
        out_specs=pl.BlockSpec((1, bm, d_h), lambda i, j: (j, i, 0)),
        out_shape=jax.ShapeDtypeStruct((3 * h, b * t, d_h), f32),
    )(x.reshape(b * t, d), W_qkv_r)
    qkv4 = qkv.reshape(3 * h, b, t, d_h)

    idx, w = pl.pallas_call(
        functools.partial(_sortnet_kern, buckets=buckets, bsz=bsz, d_h=d_h),
        grid=(b, h),
        in_specs=[
            pl.BlockSpec((1, 1, t, d_h), lambda bi, hi: (hi, bi, 0, 0)),
            pl.BlockSpec((1, 1, t, d_h), lambda bi, hi: (h + hi, bi, 0, 0)),
            pl.BlockSpec((1, 1, 2 * d_h, buckets), lambda bi, hi: (0, hi, 0, 0)),
        ],
        out_specs=[
            pl.BlockSpec((1, buckets, 1), lambda bi, hi: (bi * h + hi, 0, 0)),
            pl.BlockSpec((1, buckets, 1), lambda bi, hi: (bi * h + hi, 0, 0)),
        ],
        out_shape=[
            jax.ShapeDtypeStruct((b * h, buckets, 1), jnp.int32),
            jax.ShapeDtypeStruct((b * h, buckets, 1), f32),
        ],
    )(qkv4, qkv4, W_sort)

    attn = pl.pallas_call(
        functools.partial(_attn_kern, heads=h, buckets=buckets, bsz=bsz,
                          d_h=d_h, scale=d ** -0.5),
        grid_spec=pltpu.PrefetchScalarGridSpec(
            num_scalar_prefetch=2,
            grid=(b, h),
            in_specs=[
                pl.BlockSpec((1, 1, t, d_h), lambda bi, hi, *_: (hi, bi, 0, 0)),
                pl.BlockSpec((1, 1, t, d_h),
                             lambda bi, hi, *_: (h + hi, bi, 0, 0)),
                pl.BlockSpec((1, 1, t, d_h),
                             lambda bi, hi, *_: (2 * h + hi, bi, 0, 0)),
            ],
            out_specs=pl.BlockSpec((1, 1, t, d_h),
                                   lambda bi, hi, *_: (hi, bi, 0, 0)),
            scratch_shapes=[
                pltpu.VMEM((buckets, bsz, d_h), f32),
                pltpu.VMEM((buckets, bsz, d_h), f32),
            ],
        ),
        out_shape=jax.ShapeDtypeStruct((h, b, t, d_h), f32),
    )(idx.reshape(b * h, buckets), w.reshape(b * h, buckets),
      qkv4, qkv4, qkv4)

    bm2 = 1024
    out = pl.pallas_call(
        _out_kern,
        grid=(b * t // bm2, h),
        in_specs=[
            pl.BlockSpec((1, bm2, d_h), lambda i, c: (c, i, 0)),
            pl.BlockSpec((1, d_h, d), lambda i, c: (c, 0, 0)),
            pl.BlockSpec((1, d), lambda i, c: (0, 0)),
        ],
        out_specs=pl.BlockSpec((bm2, d), lambda i, c: (i, 0)),
        out_shape=jax.ShapeDtypeStruct((b * t, d), f32),
    )(attn.reshape(h, b * t, d_h), W_out_r, b_out_r)
    return out.reshape(b, t, d)


# speedup vs baseline: 3.5627x; 3.5627x over previous
import functools

import jax
import jax.numpy as jnp
from jax.experimental import pallas as pl
from jax.experimental.pallas import tpu as pltpu

HEADS = 16
BUCKETS = 64
GRP = 4


def _matmul_kern(x_ref, w_ref, o_ref):
    o_ref[...] = jnp.dot(x_ref[...], w_ref[...],
                         preferred_element_type=jnp.float32)


def _matmul_bias_kern(x_ref, w_ref, b_ref, o_ref):
    o_ref[...] = jnp.dot(x_ref[...], w_ref[...],
                         preferred_element_type=jnp.float32) + b_ref[...]


def _sortnet_kern(q_ref, k_ref, ws_ref, idx_ref, w_ref, *, buckets, bsz, d_h):
    for j in range(2):
        sl = slice(j * d_h, (j + 1) * d_h)
        q3 = q_ref[0, :, sl].reshape(buckets, bsz, d_h)
        k3 = k_ref[0, :, sl].reshape(buckets, bsz, d_h)
        xs = jnp.concatenate((q3.sum(axis=1), k3.sum(axis=1)), axis=-1)
        logits = jnp.dot(xs, ws_ref[0, j], preferred_element_type=jnp.float32)
        lr = jnp.where(logits >= 0, logits, 0.01 * logits)
        lrT = lr.T
        m = jnp.max(lrT, axis=0, keepdims=True)
        sumexp = jnp.sum(jnp.exp(lrT - m), axis=0, keepdims=True)
        row = jax.lax.broadcasted_iota(jnp.int32, (buckets, buckets), 0)
        idx_ref[0, 0, j:j + 1, :] = jnp.min(
            jnp.where(lrT == m, row, buckets), axis=0, keepdims=True)
        w_ref[0, 0, j:j + 1, :] = 1.0 / sumexp


def _attn_kern(idx_ref, w_ref, q_ref, k_ref, v_ref, o_ref, kg_ref, vg_ref, *,
               heads, buckets, bsz, d_h, scale, grp):
    bi, h2 = pl.program_id(0), pl.program_id(1)
    t = buckets * bsz
    m = grp * bsz
    rb = jax.lax.broadcasted_iota(jnp.int32, (m, 2 * m), 0) // bsz
    cb = jax.lax.broadcasted_iota(jnp.int32, (m, 2 * m), 1) // bsz
    mask2 = rb == jnp.where(cb >= grp, cb - grp, cb)
    ones_col = jnp.ones((2 * m, 1), dtype=jnp.float32)

    for j in range(2):
        sl = slice(j * d_h, (j + 1) * d_h)
        for u in range(buckets):
            g = idx_ref[bi, h2, j, u]
            w = w_ref[bi, h2, j, u]
            kg_ref[pl.ds(u * bsz, bsz), :] = (
                w * k_ref[0, pl.ds(g * bsz, bsz), sl])
            vg_ref[pl.ds(u * bsz, bsz), :] = (
                w * v_ref[0, pl.ds(g * bsz, bsz), sl])
        q = q_ref[0, :, sl] * scale
        k = k_ref[0, :, sl]
        v = v_ref[0, :, sl]
        K_g = kg_ref[...]
        V_g = vg_ref[...]
        dn = (((1,), (1,)), ((), ()))
        for G in range(buckets // grp):
            rs = slice(G * m, (G + 1) * m)
            KK = jnp.concatenate((K_g[rs], k[rs]), axis=0)
            VV = jnp.concatenate((V_g[rs], v[rs]), axis=0)
            VV1 = jnp.concatenate((VV, ones_col), axis=1)
            dots = jax.lax.dot_general(q[rs], KK, dn,
                                       preferred_element_type=jnp.float32)
            e = jnp.exp(jnp.where(mask2, dots, -1e30))
            pv = jnp.dot(e, VV1, preferred_element_type=jnp.float32)
            out = pv[:, :d_h] * (1.0 / pv[:, d_h:])
            o_ref[0, pl.ds(G * m, m), sl] = out


def kernel(x, W_qkv, W_out, b_out, W_sort):
    b, t, d = x.shape
    h, buckets = HEADS, BUCKETS
    d_h = d // h
    bsz = t // buckets
    f32 = jnp.float32

    bm, bn = 2048, 768
    qkv = pl.pallas_call(
        _matmul_kern,
        grid=(b * t // bm, 3 * d // bn),
        in_specs=[
            pl.BlockSpec((bm, d), lambda i, j: (i, 0)),
            pl.BlockSpec((d, bn), lambda i, j: (0, j)),
        ],
        out_specs=pl.BlockSpec((bm, bn), lambda i, j: (i, j)),
        out_shape=jax.ShapeDtypeStruct((b * t, 3 * d), f32),
    )(x.reshape(b * t, d), W_qkv).reshape(b, t, 3 * d)

    W_sort_p = W_sort.reshape(h // 2, 2, 2 * d_h, buckets)
    idx, w = pl.pallas_call(
        functools.partial(_sortnet_kern, buckets=buckets, bsz=bsz, d_h=d_h),
        grid=(b, h // 2),
        in_specs=[
            pl.BlockSpec((1, t, 2 * d_h), lambda bi, h2: (bi, 0, h2)),
            pl.BlockSpec((1, t, 2 * d_h), lambda bi, h2: (bi, 0, h // 2 + h2)),
            pl.BlockSpec((1, 2, 2 * d_h, buckets), lambda bi, h2: (h2, 0, 0, 0)),
        ],
        out_specs=[
            pl.BlockSpec((1, 1, 2, buckets), lambda bi, h2: (bi, h2, 0, 0)),
            pl.BlockSpec((1, 1, 2, buckets), lambda bi, h2: (bi, h2, 0, 0)),
        ],
        out_shape=[
            jax.ShapeDtypeStruct((b, h // 2, 2, buckets), jnp.int32),
            jax.ShapeDtypeStruct((b, h // 2, 2, buckets), f32),
        ],
    )(qkv, qkv, W_sort_p)

    attn = pl.pallas_call(
        functools.partial(_attn_kern, heads=h, buckets=buckets, bsz=bsz,
                          d_h=d_h, scale=d ** -0.5, grp=GRP),
        grid_spec=pltpu.PrefetchScalarGridSpec(
            num_scalar_prefetch=2,
            grid=(b, h // 2),
            in_specs=[
                pl.BlockSpec((1, t, 2 * d_h), lambda bi, h2, *_: (bi, 0, h2)),
                pl.BlockSpec((1, t, 2 * d_h),
                             lambda bi, h2, *_: (bi, 0, h // 2 + h2)),
                pl.BlockSpec((1, t, 2 * d_h),
                             lambda bi, h2, *_: (bi, 0, h + h2)),
            ],
            out_specs=pl.BlockSpec((1, t, 2 * d_h),
                                   lambda bi, h2, *_: (bi, 0, h2)),
            scratch_shapes=[
                pltpu.VMEM((t, d_h), f32),
                pltpu.VMEM((t, d_h), f32),
            ],
        ),
        out_shape=jax.ShapeDtypeStruct((b, t, d), f32),
    )(idx, w, qkv, qkv, qkv)

    bm2 = 1024
    out = pl.pallas_call(
        _matmul_bias_kern,
        grid=(b * t // bm2,),
        in_specs=[
            pl.BlockSpec((bm2, d), lambda i: (i, 0)),
            pl.BlockSpec((d, d), lambda i: (0, 0)),
            pl.BlockSpec((1, d), lambda i: (0, 0)),
        ],
        out_specs=pl.BlockSpec((bm2, d), lambda i: (i, 0)),
        out_shape=jax.ShapeDtypeStruct((b * t, d), f32),
    )(attn.reshape(b * t, d), W_out, b_out.reshape(1, d))
    return out.reshape(b, t, d)
